# baseline (speedup 1.0000x reference)
import jax
import jax.numpy as jnp
from jax import lax
from jax.experimental import pallas as pl
from jax.experimental.pallas import tpu as pltpu
from jax.experimental.pallas import tpu_sc as plsc

NW = 32
L = 16
CAM_PIX = 147456
CAM_WORDS = 2 * 8 * CAM_PIX
IMG_WORDS = 3538944
CH = 9216
K1_PER_TILE = CAM_WORDS // NW
K2_PER_TILE = IMG_WORDS // NW


def _tca_body(cc_ref, ch_ref, img_ref, bins_ref, mn_ref, mx_ref):
    g = pl.program_id(0)
    width = (255.0 - 0.0) / 256

    lane_off = (lax.broadcasted_iota(jnp.int32, (2304, 128), 1) % 16) * 257

    def binify(x):
        v = x * 255.0
        b = jnp.clip(jnp.floor((v - 0.0) / width), 0, 255).astype(jnp.int32)
        return b + lane_off

    bins_ref[0] = binify(cc_ref[...])
    bins_ref[1] = binify(ch_ref[...])
    img = img_ref[...]
    bmn = jnp.min(img).reshape(1, 1)
    bmx = jnp.max(img).reshape(1, 1)

    @pl.when(g == 0)
    def _():
        mn_ref[...] = bmn
        mx_ref[...] = bmx

    @pl.when(g > 0)
    def _():
        mn_ref[...] = jnp.minimum(mn_ref[...], bmn)
        mx_ref[...] = jnp.maximum(mx_ref[...], bmx)


def _k1_body(bins_hbm, hist_out, dbuf, hscr, fbuf, s0, s1):
    wid = lax.axis_index("s") * 2 + lax.axis_index("c")
    zeros16 = jnp.zeros((L,), jnp.float32)
    ones16 = jnp.ones((L,), jnp.float32)

    def zero_hist(i, _):
        for u in range(8):
            hscr[pl.ds((i * 8 + u) * L, L)] = zeros16
        return 0
    lax.fori_loop(0, 33, zero_hist, 0)

    off = wid * K1_PER_TILE
    sems = (s0, s1)
    nch = K1_PER_TILE // CH

    def start(k):
        return pltpu.async_copy(
            bins_hbm.at[pl.ds(off + k * CH, CH)],
            dbuf.at[pl.ds((k % 2) * CH, CH)], sems[k % 2])

    handles = {0: start(0)}
    for k in range(nch):
        handles[k].wait()
        if k + 1 < nch:
            handles[k + 1] = start(k + 1)
        base = (k % 2) * CH

        @plsc.parallel_loop(0, CH // (L * 8))
        def inner(i):
            for u in range(8):
                b = dbuf[pl.ds(base + (i * 8 + u) * L, L)]
                plsc.addupdate_scatter(hscr, [b], ones16)

    def red(g, _):
        acc = zeros16
        for l in range(L):
            acc = acc + hscr[pl.ds(l * 257 + g * L, L)]
        fbuf[pl.ds(g * L, L)] = acc
        return 0
    lax.fori_loop(0, 16, red, 0)
    pltpu.sync_copy(fbuf, hist_out.at[wid])


def _tcb_body(tc_ref, th_ref, mn_ref, mx_ref, cc_ref, ch_ref,
              img_ref, pre_ref):
    b = pl.program_id(0)
    tc = tc_ref[b]
    th = th_ref[b]
    mn0 = mn_ref[0]
    mx = mx_ref[0]
    mn = jnp.where(mn0 == 0, mn0 + 0.001, mn0)
    wd = (mx - mn) / 256
    cc = cc_ref[0]
    ch = ch_ref[0]
    mcomb = (jnp.where(cc * 255.0 > tc, 512, 0)
             + jnp.where(ch * 255.0 > th, 256, 0)).astype(jnp.int32)
    lane_off = (lax.broadcasted_iota(jnp.int32, (1152, 128), 1) % 16) * 3073
    for c in range(3):
        x = img_ref[0, c]
        valid = (x >= mn) & (x <= mx)
        idx = jnp.clip(jnp.floor((x - mn) / wd), 0, 255).astype(jnp.int32)
        pre_ref[0, c] = jnp.where(valid, idx + mcomb + c * 1024, 3072) + lane_off


def _k2_body(pre_hbm, hist_out, dbuf, hscr, fbuf, s0, s1):
    wid = lax.axis_index("s") * 2 + lax.axis_index("c")
    zeros16 = jnp.zeros((L,), jnp.float32)
    ones16 = jnp.ones((L,), jnp.float32)

    def zero_hist(i, _):
        for u in range(8):
            hscr[pl.ds((i * 8 + u) * L, L)] = zeros16
        return 0
    lax.fori_loop(0, 385, zero_hist, 0)

    off = wid * K2_PER_TILE
    sems = (s0, s1)
    nch = K2_PER_TILE // CH

    def start(k):
        return pltpu.async_copy(
            pre_hbm.at[pl.ds(off + k * CH, CH)],
            dbuf.at[pl.ds((k % 2) * CH, CH)], sems[k % 2])

    handles = {0: start(0)}
    for k in range(nch):
        handles[k].wait()
        if k + 1 < nch:
            handles[k + 1] = start(k + 1)
        base = (k % 2) * CH

        @plsc.parallel_loop(0, CH // (L * 8))
        def inner(i):
            for u in range(8):
                b = dbuf[pl.ds(base + (i * 8 + u) * L, L)]
                plsc.addupdate_scatter(hscr, [b], ones16)

    def red(g, _):
        acc = zeros16
        for l in range(L):
            acc = acc + hscr[pl.ds(l * 3073 + g * L, L)]
        fbuf[pl.ds(g * L, L)] = acc
        return 0
    lax.fori_loop(0, 192, red, 0)
    pltpu.sync_copy(fbuf, hist_out.at[wid])


def _ce_body(hp_ref, o_ref):
    eps = 1e-10
    H = jnp.sum(hp_ref[...].reshape(32, 12, 256), axis=0)

    def row(i):
        return H[i]

    fc = [row(4 * c + 2) + row(4 * c + 3) for c in range(3)]
    bc = [row(4 * c + 0) + row(4 * c + 1) for c in range(3)]
    fh = [row(4 * c + 1) + row(4 * c + 3) for c in range(3)]
    bh = [row(4 * c + 0) + row(4 * c + 2) for c in range(3)]

    def tot(v):
        return jnp.sum(v[0]) + jnp.sum(v[1]) + jnp.sum(v[2])

    def prep(v):
        s = tot(v)
        return [jnp.clip(x / s, eps, None) for x in v]

    pfc, pbc, pfh, pbh = prep(fc), prep(bc), prep(fh), prep(bh)
    lfh = [jnp.log(x) for x in pfh]
    lbh = [jnp.log(x) for x in pbh]

    def ce(pa, lb):
        return -(jnp.sum(pa[0] * lb[0]) + jnp.sum(pa[1] * lb[1])
                 + jnp.sum(pa[2] * lb[2]))

    ce_pos = ce(pfc, lfh) + ce(pbc, lbh)
    ce_neg = -(ce(pfc, lbh) + ce(pbc, lfh))
    o_ref[...] = jnp.full((8, 128), 1.0 * ce_pos + 0.5 * ce_neg)


def _otsu_threshold(hist):
    prob = hist / jnp.sum(hist, axis=2, keepdims=True)
    cum_prob = jnp.cumsum(prob, axis=2)
    cum_mean = jnp.cumsum(prob * jnp.arange(256, dtype=jnp.float32)[None, None, :], axis=2)
    global_mean = cum_mean[:, :, -1:]
    numerator = (global_mean * cum_prob - cum_mean) ** 2
    denominator = cum_prob * (1.0 - cum_prob)
    between_class_variance = numerator / denominator
    return jnp.argmax(between_class_variance, axis=2)


def kernel(cam_cln, cam_haz, img_haz):
    mesh = plsc.VectorSubcoreMesh(core_axis_name="c", subcore_axis_name="s")
    sc_params = pltpu.CompilerParams(needs_layout_passes=False)

    cc2 = cam_cln.reshape(9216, 128)
    ch2 = cam_haz.reshape(9216, 128)
    img2 = img_haz.reshape(27648, 128)

    cam_bins, mn0, mx0 = pl.pallas_call(
        _tca_body,
        grid=(4,),
        in_specs=[
            pl.BlockSpec((2304, 128), lambda g: (g, 0)),
            pl.BlockSpec((2304, 128), lambda g: (g, 0)),
            pl.BlockSpec((6912, 128), lambda g: (g, 0)),
        ],
        out_specs=[
            pl.BlockSpec((2, 2304, 128), lambda g: (0, g, 0)),
            pl.BlockSpec((1, 1), lambda g: (0, 0)),
            pl.BlockSpec((1, 1), lambda g: (0, 0)),
        ],
        out_shape=[jax.ShapeDtypeStruct((2, 9216, 128), jnp.int32),
                   jax.ShapeDtypeStruct((1, 1), jnp.float32),
                   jax.ShapeDtypeStruct((1, 1), jnp.float32)],
    )(cc2, ch2, img2)

    k1 = pl.kernel(
        _k1_body,
        out_type=[jax.ShapeDtypeStruct((NW, 256), jnp.float32)],
        mesh=mesh,
        scratch_types=[pltpu.VMEM((2 * CH,), jnp.int32),
                       pltpu.VMEM((4224,), jnp.float32),
                       pltpu.VMEM((256,), jnp.float32),
                       pltpu.SemaphoreType.DMA,
                       pltpu.SemaphoreType.DMA],
        compiler_params=sc_params,
    )
    hist_part = k1(cam_bins.reshape(-1))[0]

    hist16 = hist_part.reshape(16, 2, 256).sum(axis=1)
    t_all = _otsu_threshold(hist16.reshape(16, 1, 256)).reshape(16).astype(jnp.float32)
    t_cln_f = t_all[:8]
    t_haz_f = t_all[8:]

    mn1 = mn0.reshape(1)
    mx1 = mx0.reshape(1)

    cc3 = cam_cln.reshape(8, 1152, 128)
    ch3 = cam_haz.reshape(8, 1152, 128)
    img4 = img_haz.reshape(8, 3, 1152, 128)

    pre = pl.pallas_call(
        _tcb_body,
        grid=(8,),
        in_specs=[
            pl.BlockSpec(memory_space=pltpu.SMEM),
            pl.BlockSpec(memory_space=pltpu.SMEM),
            pl.BlockSpec(memory_space=pltpu.SMEM),
            pl.BlockSpec(memory_space=pltpu.SMEM),
            pl.BlockSpec((1, 1152, 128), lambda b: (b, 0, 0)),
            pl.BlockSpec((1, 1152, 128), lambda b: (b, 0, 0)),
            pl.BlockSpec((1, 3, 1152, 128), lambda b: (b, 0, 0, 0)),
        ],
        out_specs=pl.BlockSpec((1, 3, 1152, 128), lambda b: (b, 0, 0, 0)),
        out_shape=jax.ShapeDtypeStruct((8, 3, 1152, 128), jnp.int32),
    )(t_cln_f, t_haz_f, mn1, mx1, cc3, ch3, img4)

    k2 = pl.kernel(
        _k2_body,
        out_type=[jax.ShapeDtypeStruct((NW, 3072), jnp.float32)],
        mesh=mesh,
        scratch_types=[pltpu.VMEM((2 * CH,), jnp.int32),
                       pltpu.VMEM((49280,), jnp.float32),
                       pltpu.VMEM((3072,), jnp.float32),
                       pltpu.SemaphoreType.DMA,
                       pltpu.SemaphoreType.DMA],
        compiler_params=sc_params,
    )
    hist2_part = k2(pre.reshape(-1))[0]

    out = pl.pallas_call(
        _ce_body,
        out_shape=jax.ShapeDtypeStruct((8, 128), jnp.float32),
    )(hist2_part)
    return out[0, 0]
